# Initial kernel scaffold; baseline (speedup 1.0000x reference)
import functools

import jax
import jax.numpy as jnp
from jax import lax
from jax.experimental import pallas as pl
from jax.experimental.pallas import tpu as pltpu
from jax.experimental.pallas import tpu_sc as plsc

N = 10000
NFEAT = 128
NHID = 256
NCLASS = 64
E = 320000

NC, NS = 2, 16
NPAD = 10240
EPAD = 327680
CH = 128
RPT = NPAD // NS
DEGW = 16
BN = 2000


def _sc_mesh():
    return plsc.VectorSubcoreMesh(
        core_axis_name="c", subcore_axis_name="s", num_cores=NC, num_subcores=NS
    )


@functools.partial(
    pl.kernel,
    out_type=jax.ShapeDtypeStruct((NC, NPAD, DEGW), jnp.float32),
    mesh=_sc_mesh(),
    scratch_types=[
        pltpu.VMEM((1, CH), jnp.int32),
        pltpu.VMEM((CH, DEGW), jnp.float32),
        pltpu.VMEM_SHARED((NPAD, DEGW), jnp.float32),
    ],
)
def _deg_sc(colp_hbm, ones_hbm, zeros_hbm, out_hbm, idx_c, ones_v, acc_sh):
    c = lax.axis_index("c")
    s = lax.axis_index("s")
    pltpu.sync_copy(zeros_hbm, acc_sh.at[pl.ds(s * RPT, RPT)])
    pltpu.sync_copy(ones_hbm, ones_v)
    plsc.subcore_barrier()
    base = (c * NS + s) * (EPAD // (NC * NS))
    nch = (EPAD // (NC * NS)) // CH

    def body(g, carry):
        off = base + g * CH
        pltpu.sync_copy(colp_hbm.at[pl.ds(off, CH)], idx_c.at[0])
        pltpu.sync_copy(ones_v, acc_sh.at[idx_c.at[0]], add=True)
        return carry

    lax.fori_loop(0, nch, body, 0)
    plsc.subcore_barrier()
    pltpu.sync_copy(
        acc_sh.at[pl.ds(s * RPT, RPT)], out_hbm.at[c, pl.ds(s * RPT, RPT)]
    )


def _make_prop(width, colsplit):
    edges_per_tile = EPAD // NS if colsplit else EPAD // (NC * NS)
    nch = edges_per_tile // CH

    @functools.partial(
        pl.kernel,
        out_type=jax.ShapeDtypeStruct((NC, NPAD, width), jnp.float32),
        mesh=_sc_mesh(),
        scratch_types=[
            pltpu.VMEM((CH,), jnp.int32),
            pltpu.VMEM((1, CH), jnp.int32),
            pltpu.VMEM((CH, width), jnp.float32),
            pltpu.VMEM_SHARED((NPAD, width), jnp.float32),
        ],
    )
    def prop(u_hbm, rowp_hbm, colp_hbm, zeros_hbm, out_hbm,
             idx_g, idx_c, rows_v, acc_sh):
        c = lax.axis_index("c")
        s = lax.axis_index("s")
        pltpu.sync_copy(zeros_hbm, acc_sh.at[pl.ds(s * RPT, RPT)])
        plsc.subcore_barrier()
        base = (s if colsplit else c * NS + s) * edges_per_tile

        def body(g, carry):
            off = base + g * CH
            pltpu.sync_copy(rowp_hbm.at[pl.ds(off, CH)], idx_g)
            if colsplit:
                for i in range(CH // 16):
                    sl = pl.ds(i * 16, 16)
                    idx_g[sl] = idx_g[sl] + c * N
            pltpu.sync_copy(u_hbm.at[idx_g], rows_v)
            pltpu.sync_copy(colp_hbm.at[pl.ds(off, CH)], idx_c.at[0])
            pltpu.sync_copy(rows_v, acc_sh.at[idx_c.at[0]], add=True)
            return carry

        lax.fori_loop(0, nch, body, 0)
        plsc.subcore_barrier()
        pltpu.sync_copy(
            acc_sh.at[pl.ds(s * RPT, RPT)], out_hbm.at[c, pl.ds(s * RPT, RPT)]
        )

    return prop


_prop128_edge = _make_prop(128, colsplit=False)
_prop128_col = _make_prop(128, colsplit=True)
_prop64_edge = _make_prop(64, colsplit=False)


def _dis_u0_body(d0_ref, d1_ref, x_ref, dis_ref, u0_ref):
    deg = d0_ref[...] + d1_ref[...] + 1.0
    dis = lax.rsqrt(deg)
    dis_ref[...] = dis
    u0_ref[...] = dis * x_ref[...]


def _dis_u0(d0, d1, x):
    grid = (N // BN,)
    return pl.pallas_call(
        _dis_u0_body,
        grid=grid,
        in_specs=[
            pl.BlockSpec((BN, 1), lambda i: (i, 0)),
            pl.BlockSpec((BN, 1), lambda i: (i, 0)),
            pl.BlockSpec((BN, NFEAT), lambda i: (i, 0)),
        ],
        out_specs=[
            pl.BlockSpec((BN, 1), lambda i: (i, 0)),
            pl.BlockSpec((BN, NFEAT), lambda i: (i, 0)),
        ],
        out_shape=[
            jax.ShapeDtypeStruct((N, 1), jnp.float32),
            jax.ShapeDtypeStruct((N, NFEAT), jnp.float32),
        ],
    )(d0, d1, x)


def _layer0_body(s0a_ref, s0b_ref, u0_ref, dis_ref, w_ref, sc_ref, bi_ref,
                 u1_ref):
    dis = dis_ref[...]
    z = dis * (s0a_ref[...] + s0b_ref[...] + u0_ref[...])
    y = jnp.dot(z, w_ref[...], preferred_element_type=jnp.float32)
    h = jnp.maximum(y * sc_ref[...] + bi_ref[...], 0.0)
    u1_ref[0] = dis * h


def _layer0(s0a, s0b, u0, dis, w0t, scale0, bias0):
    grid = (N // BN, 2)
    return pl.pallas_call(
        _layer0_body,
        grid=grid,
        in_specs=[
            pl.BlockSpec((BN, NFEAT), lambda i, j: (i, 0)),
            pl.BlockSpec((BN, NFEAT), lambda i, j: (i, 0)),
            pl.BlockSpec((BN, NFEAT), lambda i, j: (i, 0)),
            pl.BlockSpec((BN, 1), lambda i, j: (i, 0)),
            pl.BlockSpec((NFEAT, 128), lambda i, j: (0, j)),
            pl.BlockSpec((1, 128), lambda i, j: (0, j)),
            pl.BlockSpec((1, 128), lambda i, j: (0, j)),
        ],
        out_specs=pl.BlockSpec((1, BN, 128), lambda i, j: (j, i, 0)),
        out_shape=jax.ShapeDtypeStruct((2, N, 128), jnp.float32),
    )(s0a, s0b, u0, dis, w0t, scale0, bias0)


def _layer12_body(s1a_ref, s1b_ref, u1a_ref, u1b_ref, dis_ref, wa_ref, wb_ref,
                  sc_ref, bi_ref, w2_ref, z2_ref):
    dis = dis_ref[...]
    za = dis * (s1a_ref[...] + u1a_ref[...])
    zb = dis * (s1b_ref[...] + u1b_ref[...])
    y = jnp.dot(za, wa_ref[...], preferred_element_type=jnp.float32)
    y += jnp.dot(zb, wb_ref[...], preferred_element_type=jnp.float32)
    h2 = jnp.maximum(y * sc_ref[...] + bi_ref[...], 0.0)
    z2_ref[...] = dis * jnp.dot(h2, w2_ref[...],
                                preferred_element_type=jnp.float32)


def _layer12(s1a, s1b, u1a, u1b, dis, w1ta, w1tb, scale1, bias1, w2t):
    grid = (N // BN,)
    return pl.pallas_call(
        _layer12_body,
        grid=grid,
        in_specs=[
            pl.BlockSpec((BN, 128), lambda i: (i, 0)),
            pl.BlockSpec((BN, 128), lambda i: (i, 0)),
            pl.BlockSpec((BN, 128), lambda i: (i, 0)),
            pl.BlockSpec((BN, 128), lambda i: (i, 0)),
            pl.BlockSpec((BN, 1), lambda i: (i, 0)),
            pl.BlockSpec((128, NHID), lambda i: (0, 0)),
            pl.BlockSpec((128, NHID), lambda i: (0, 0)),
            pl.BlockSpec((1, NHID), lambda i: (0, 0)),
            pl.BlockSpec((1, NHID), lambda i: (0, 0)),
            pl.BlockSpec((NHID, NCLASS), lambda i: (0, 0)),
        ],
        out_specs=pl.BlockSpec((BN, NCLASS), lambda i: (i, 0)),
        out_shape=jax.ShapeDtypeStruct((N, NCLASS), jnp.float32),
    )(s1a, s1b, u1a, u1b, dis, w1ta, w1tb, scale1, bias1, w2t)


def _final_body(s2a_ref, s2b_ref, z2_ref, dis_ref, b2_ref, out_ref):
    out_ref[...] = (
        dis_ref[...] * (s2a_ref[...] + s2b_ref[...] + z2_ref[...]) + b2_ref[...]
    )


def _final(s2a, s2b, z2, dis, b2):
    grid = (N // BN,)
    return pl.pallas_call(
        _final_body,
        grid=grid,
        in_specs=[
            pl.BlockSpec((BN, NCLASS), lambda i: (i, 0)),
            pl.BlockSpec((BN, NCLASS), lambda i: (i, 0)),
            pl.BlockSpec((BN, NCLASS), lambda i: (i, 0)),
            pl.BlockSpec((BN, 1), lambda i: (i, 0)),
            pl.BlockSpec((1, NCLASS), lambda i: (0, 0)),
        ],
        out_specs=pl.BlockSpec((BN, NCLASS), lambda i: (i, 0)),
        out_shape=jax.ShapeDtypeStruct((N, NCLASS), jnp.float32),
    )(s2a, s2b, z2, dis, b2)


def kernel(x, edge_idx, W0, b0, g0, be0, W1, b1, g1, be1, W2, b2):
    fill = EPAD - E
    rowp = jnp.concatenate([edge_idx[0], jnp.zeros((fill,), jnp.int32)])
    colp = jnp.concatenate([edge_idx[1], jnp.full((fill,), N, jnp.int32)])

    ones = jnp.ones((CH, DEGW), jnp.float32)
    zeros_deg = jnp.zeros((RPT, DEGW), jnp.float32)
    zeros128 = jnp.zeros((RPT, 128), jnp.float32)
    zeros64 = jnp.zeros((RPT, 64), jnp.float32)

    c0 = 1.0 / jnp.sqrt(jnp.float32(1.0 + 1e-5))
    scale0 = (g0 * c0)[None, :]
    bias0 = (b0 * g0 * c0 + be0)[None, :]
    scale1 = (g1 * c0)[None, :]
    bias1 = (b1 * g1 * c0 + be1)[None, :]
    w0t = W0.T
    w1t = W1.T
    w2t = W2.T

    degp = _deg_sc(colp, ones, zeros_deg)
    d0 = degp[0, :N, :1]
    d1 = degp[1, :N, :1]
    dis, u0 = _dis_u0(d0, d1, x)

    s0 = _prop128_edge(u0, rowp, colp, zeros128)
    u1 = _layer0(s0[0, :N], s0[1, :N], u0, dis, w0t, scale0, bias0)

    s1 = _prop128_col(u1.reshape(2 * N, 128), rowp, colp, zeros128)
    z2 = _layer12(s1[0, :N], s1[1, :N], u1[0], u1[1], dis,
                  w1t[:128], w1t[128:], scale1, bias1, w2t)

    s2 = _prop64_edge(z2, rowp, colp, zeros64)
    return _final(s2[0, :N], s2[1, :N], z2, dis, b2)


# speedup vs baseline: 6.5878x; 6.5878x over previous
import functools

import jax
import jax.numpy as jnp
from jax import lax
from jax.experimental import pallas as pl
from jax.experimental.pallas import tpu as pltpu
from jax.experimental.pallas import tpu_sc as plsc

N = 10000
NFEAT = 128
NHID = 256
NCLASS = 64
E = 320000

NC, NS = 2, 16
NPAD = 10240
EPAD = 327680
CH = 128
RPT = NPAD // NS
DEGW = 16
BN = 2000


def _sc_mesh():
    return plsc.VectorSubcoreMesh(
        core_axis_name="c", subcore_axis_name="s", num_cores=NC, num_subcores=NS
    )


@functools.partial(
    pl.kernel,
    out_type=jax.ShapeDtypeStruct((NC, NPAD, DEGW), jnp.float32),
    mesh=_sc_mesh(),
    scratch_types=[
        pltpu.VMEM((1, CH), jnp.int32),
        pltpu.VMEM((CH, DEGW), jnp.float32),
        pltpu.VMEM_SHARED((NPAD, DEGW), jnp.float32),
    ],
)
def _deg_sc(colp_hbm, ones_hbm, zeros_hbm, out_hbm, idx_c, ones_v, acc_sh):
    c = lax.axis_index("c")
    s = lax.axis_index("s")
    pltpu.sync_copy(zeros_hbm, acc_sh.at[pl.ds(s * RPT, RPT)])
    pltpu.sync_copy(ones_hbm, ones_v)
    plsc.subcore_barrier()
    base = (c * NS + s) * (EPAD // (NC * NS))
    nch = (EPAD // (NC * NS)) // CH

    def body(g, carry):
        off = base + g * CH
        pltpu.sync_copy(colp_hbm.at[pl.ds(off, CH)], idx_c.at[0])
        pltpu.sync_copy(ones_v, acc_sh.at[idx_c.at[0]], add=True)
        return carry

    lax.fori_loop(0, nch, body, 0)
    plsc.subcore_barrier()
    pltpu.sync_copy(
        acc_sh.at[pl.ds(s * RPT, RPT)], out_hbm.at[c, pl.ds(s * RPT, RPT)]
    )


def _make_prop(width, colsplit):
    edges_per_tile = EPAD // NS if colsplit else EPAD // (NC * NS)
    nch = edges_per_tile // CH

    @functools.partial(
        pl.kernel,
        out_type=jax.ShapeDtypeStruct((NC, NPAD, width), jnp.float32),
        mesh=_sc_mesh(),
        scratch_types=[
            pltpu.VMEM((CH,), jnp.int32),
            pltpu.VMEM((1, CH), jnp.int32),
            pltpu.VMEM((CH, width), jnp.float32),
            pltpu.VMEM_SHARED((NPAD, width), jnp.float32),
        ],
    )
    def prop(u_hbm, rowp_hbm, colp_hbm, zeros_hbm, out_hbm,
             idx_g, idx_c, rows_v, acc_sh):
        c = lax.axis_index("c")
        s = lax.axis_index("s")
        pltpu.sync_copy(zeros_hbm, acc_sh.at[pl.ds(s * RPT, RPT)])
        plsc.subcore_barrier()
        base = (s if colsplit else c * NS + s) * edges_per_tile

        def body(g, carry):
            off = base + g * CH
            pltpu.sync_copy(rowp_hbm.at[pl.ds(off, CH)], idx_g)
            if colsplit:
                for i in range(CH // 16):
                    sl = pl.ds(i * 16, 16)
                    idx_g[sl] = idx_g[sl] + c * N
            pltpu.sync_copy(u_hbm.at[idx_g], rows_v)
            pltpu.sync_copy(colp_hbm.at[pl.ds(off, CH)], idx_c.at[0])
            pltpu.sync_copy(rows_v, acc_sh.at[idx_c.at[0]], add=True)
            return carry

        lax.fori_loop(0, nch, body, 0)
        plsc.subcore_barrier()
        pltpu.sync_copy(
            acc_sh.at[pl.ds(s * RPT, RPT)], out_hbm.at[c, pl.ds(s * RPT, RPT)]
        )

    return prop


_prop128_edge = _make_prop(128, colsplit=False)
_prop128_col = _make_prop(128, colsplit=True)


def _dis_u0_body(d0_ref, d1_ref, x_ref, dis_ref, u0_ref):
    deg = d0_ref[...] + d1_ref[...] + 1.0
    dis = lax.rsqrt(deg)
    dis_ref[...] = dis
    u0_ref[...] = dis * x_ref[...]


def _dis_u0(d0, d1, x):
    grid = (N // BN,)
    return pl.pallas_call(
        _dis_u0_body,
        grid=grid,
        in_specs=[
            pl.BlockSpec((BN, 1), lambda i: (i, 0)),
            pl.BlockSpec((BN, 1), lambda i: (i, 0)),
            pl.BlockSpec((BN, NFEAT), lambda i: (i, 0)),
        ],
        out_specs=[
            pl.BlockSpec((BN, 1), lambda i: (i, 0)),
            pl.BlockSpec((BN, NFEAT), lambda i: (i, 0)),
        ],
        out_shape=[
            jax.ShapeDtypeStruct((N, 1), jnp.float32),
            jax.ShapeDtypeStruct((N, NFEAT), jnp.float32),
        ],
    )(d0, d1, x)


def _layer0_body(s0a_ref, s0b_ref, u0_ref, dis_ref, w_ref, sc_ref, bi_ref,
                 u1_ref):
    dis = dis_ref[...]
    z = dis * (s0a_ref[...] + s0b_ref[...] + u0_ref[...])
    y = jnp.dot(z, w_ref[...], preferred_element_type=jnp.float32)
    h = jnp.maximum(y * sc_ref[...] + bi_ref[...], 0.0)
    u1_ref[0] = dis * h


def _layer0(s0a, s0b, u0, dis, w0t, scale0, bias0):
    grid = (N // BN, 2)
    return pl.pallas_call(
        _layer0_body,
        grid=grid,
        in_specs=[
            pl.BlockSpec((BN, NFEAT), lambda i, j: (i, 0)),
            pl.BlockSpec((BN, NFEAT), lambda i, j: (i, 0)),
            pl.BlockSpec((BN, NFEAT), lambda i, j: (i, 0)),
            pl.BlockSpec((BN, 1), lambda i, j: (i, 0)),
            pl.BlockSpec((NFEAT, 128), lambda i, j: (0, j)),
            pl.BlockSpec((1, 128), lambda i, j: (0, j)),
            pl.BlockSpec((1, 128), lambda i, j: (0, j)),
        ],
        out_specs=pl.BlockSpec((1, BN, 128), lambda i, j: (j, i, 0)),
        out_shape=jax.ShapeDtypeStruct((2, N, 128), jnp.float32),
    )(s0a, s0b, u0, dis, w0t, scale0, bias0)


def _layer12_body(s1a_ref, s1b_ref, u1a_ref, u1b_ref, dis_ref, wa_ref, wb_ref,
                  sc_ref, bi_ref, w2_ref, z2_ref):
    dis = dis_ref[...]
    za = dis * (s1a_ref[...] + u1a_ref[...])
    zb = dis * (s1b_ref[...] + u1b_ref[...])
    y = jnp.dot(za, wa_ref[...], preferred_element_type=jnp.float32)
    y += jnp.dot(zb, wb_ref[...], preferred_element_type=jnp.float32)
    h2 = jnp.maximum(y * sc_ref[...] + bi_ref[...], 0.0)
    z2_ref[...] = dis * jnp.dot(h2, w2_ref[...],
                                preferred_element_type=jnp.float32)


def _layer12(s1a, s1b, u1a, u1b, dis, w1ta, w1tb, scale1, bias1, w2t):
    grid = (N // BN,)
    return pl.pallas_call(
        _layer12_body,
        grid=grid,
        in_specs=[
            pl.BlockSpec((BN, 128), lambda i: (i, 0)),
            pl.BlockSpec((BN, 128), lambda i: (i, 0)),
            pl.BlockSpec((BN, 128), lambda i: (i, 0)),
            pl.BlockSpec((BN, 128), lambda i: (i, 0)),
            pl.BlockSpec((BN, 1), lambda i: (i, 0)),
            pl.BlockSpec((128, NHID), lambda i: (0, 0)),
            pl.BlockSpec((128, NHID), lambda i: (0, 0)),
            pl.BlockSpec((1, NHID), lambda i: (0, 0)),
            pl.BlockSpec((1, NHID), lambda i: (0, 0)),
            pl.BlockSpec((NHID, 128), lambda i: (0, 0)),
        ],
        out_specs=pl.BlockSpec((BN, 128), lambda i: (i, 0)),
        out_shape=jax.ShapeDtypeStruct((N, 128), jnp.float32),
    )(s1a, s1b, u1a, u1b, dis, w1ta, w1tb, scale1, bias1, w2t)


def _final_body(s2a_ref, s2b_ref, z2_ref, dis_ref, b2_ref, out_ref):
    out_ref[...] = (
        dis_ref[...] * (s2a_ref[...] + s2b_ref[...] + z2_ref[...]) + b2_ref[...]
    )


def _final(s2a, s2b, z2, dis, b2):
    grid = (N // BN,)
    return pl.pallas_call(
        _final_body,
        grid=grid,
        in_specs=[
            pl.BlockSpec((BN, NCLASS), lambda i: (i, 0)),
            pl.BlockSpec((BN, NCLASS), lambda i: (i, 0)),
            pl.BlockSpec((BN, NCLASS), lambda i: (i, 0)),
            pl.BlockSpec((BN, 1), lambda i: (i, 0)),
            pl.BlockSpec((1, NCLASS), lambda i: (0, 0)),
        ],
        out_specs=pl.BlockSpec((BN, NCLASS), lambda i: (i, 0)),
        out_shape=jax.ShapeDtypeStruct((N, NCLASS), jnp.float32),
    )(s2a, s2b, z2, dis, b2)


def kernel(x, edge_idx, W0, b0, g0, be0, W1, b1, g1, be1, W2, b2):
    fill = EPAD - E
    rowp = jnp.concatenate([edge_idx[0], jnp.zeros((fill,), jnp.int32)])
    colp = jnp.concatenate([edge_idx[1], jnp.full((fill,), N, jnp.int32)])

    ones = jnp.ones((CH, DEGW), jnp.float32)
    zeros_deg = jnp.zeros((RPT, DEGW), jnp.float32)
    zeros128 = jnp.zeros((RPT, 128), jnp.float32)

    c0 = 1.0 / jnp.sqrt(jnp.float32(1.0 + 1e-5))
    scale0 = (g0 * c0)[None, :]
    bias0 = (b0 * g0 * c0 + be0)[None, :]
    scale1 = (g1 * c0)[None, :]
    bias1 = (b1 * g1 * c0 + be1)[None, :]
    w0t = W0.T
    w1t = W1.T
    w2t = jnp.pad(W2.T, ((0, 0), (0, 128 - NCLASS)))

    degp = _deg_sc(colp, ones, zeros_deg)
    d0 = degp[0, :N, :1]
    d1 = degp[1, :N, :1]
    dis, u0 = _dis_u0(d0, d1, x)

    s0 = _prop128_edge(u0, rowp, colp, zeros128)
    u1 = _layer0(s0[0, :N], s0[1, :N], u0, dis, w0t, scale0, bias0)

    s1 = _prop128_col(u1.reshape(2 * N, 128), rowp, colp, zeros128)
    z2 = _layer12(s1[0, :N], s1[1, :N], u1[0], u1[1], dis,
                  w1t[:128], w1t[128:], scale1, bias1, w2t)

    s2 = _prop128_edge(z2, rowp, colp, zeros128)
    return _final(s2[0, :N, :NCLASS], s2[1, :N, :NCLASS], z2[:, :NCLASS],
                  dis, b2[None, :])


# speedup vs baseline: 7.5233x; 1.1420x over previous
import functools

import jax
import jax.numpy as jnp
from jax import lax
from jax.experimental import pallas as pl
from jax.experimental.pallas import tpu as pltpu
from jax.experimental.pallas import tpu_sc as plsc

N = 10000
NFEAT = 128
NHID = 256
NCLASS = 64
E = 320000

NC, NS = 2, 16
NPAD = 10240
EPAD = 327680
CH = 128
RPT = NPAD // NS
DEGW = 16
BN = 2000


def _sc_mesh():
    return plsc.VectorSubcoreMesh(
        core_axis_name="c", subcore_axis_name="s", num_cores=NC, num_subcores=NS
    )


KB = 2
NCHUNK = EPAD // CH


@functools.partial(
    pl.kernel,
    out_type=jax.ShapeDtypeStruct((NC, NPAD, DEGW), jnp.float32),
    mesh=_sc_mesh(),
    scratch_types=[
        pltpu.VMEM((KB, CH), jnp.int32),
        pltpu.VMEM((CH, DEGW), jnp.float32),
        pltpu.VMEM_SHARED((NPAD, DEGW), jnp.float32),
        pltpu.SemaphoreType.DMA,
        pltpu.SemaphoreType.DMA,
    ],
)
def _deg_sc(colp_hbm, ones_hbm, zeros_hbm, out_hbm, idx_v, ones_v, acc_sh,
            isem, ssem):
    c = lax.axis_index("c")
    s = lax.axis_index("s")
    pltpu.sync_copy(zeros_hbm, acc_sh.at[pl.ds(s * RPT, RPT)])
    pltpu.sync_copy(ones_hbm, ones_v)
    plsc.subcore_barrier()
    nch = NCHUNK // (NC * NS)
    base = (c * NS + s) * nch

    def body(t, carry):
        ch0 = base + t * KB
        ids = [
            pltpu.async_copy(colp_hbm.at[pl.ds((ch0 + b) * CH, CH)],
                             idx_v.at[b], isem)
            for b in range(KB)
        ]
        for d in ids:
            d.wait()
        sds = [
            pltpu.async_copy(ones_v, acc_sh.at[idx_v.at[b]], ssem, add=True)
            for b in range(KB)
        ]
        for d in sds:
            d.wait()
        return carry

    lax.fori_loop(0, nch // KB, body, 0)
    plsc.subcore_barrier()
    pltpu.sync_copy(
        acc_sh.at[pl.ds(s * RPT, RPT)], out_hbm.at[c, pl.ds(s * RPT, RPT)]
    )


def _make_prop(width, colsplit):
    nch = NCHUNK // NS if colsplit else NCHUNK // (NC * NS)

    @functools.partial(
        pl.kernel,
        out_type=jax.ShapeDtypeStruct((NC, NPAD, width), jnp.float32),
        mesh=_sc_mesh(),
        scratch_types=[
            pltpu.VMEM((2 * KB, CH), jnp.int32),
            pltpu.VMEM((KB, CH, width), jnp.float32),
            pltpu.VMEM_SHARED((NPAD, width), jnp.float32),
            pltpu.SemaphoreType.DMA,
            pltpu.SemaphoreType.DMA,
            pltpu.SemaphoreType.DMA,
            pltpu.SemaphoreType.DMA,
        ],
    )
    def prop(u_hbm, rowp_hbm, colp_hbm, zeros_hbm, out_hbm,
             idx_v, rows_v, acc_sh, isem, gsem0, gsem1, ssem):
        gsems = [gsem0, gsem1]
        c = lax.axis_index("c")
        s = lax.axis_index("s")
        pltpu.sync_copy(zeros_hbm, acc_sh.at[pl.ds(s * RPT, RPT)])
        plsc.subcore_barrier()
        base = (s if colsplit else c * NS + s) * nch

        def body(t, carry):
            ch0 = base + t * KB
            ids = []
            for b in range(KB):
                off = (ch0 + b) * CH
                ids.append(pltpu.async_copy(rowp_hbm.at[pl.ds(off, CH)],
                                            idx_v.at[2 * b], isem))
                ids.append(pltpu.async_copy(colp_hbm.at[pl.ds(off, CH)],
                                            idx_v.at[2 * b + 1], isem))
            for d in ids:
                d.wait()
            gds = []
            for b in range(KB):
                if colsplit:
                    for i in range(CH // 16):
                        sl = pl.ds(i * 16, 16)
                        idx_v[2 * b, sl] = idx_v[2 * b, sl] + c * N
                gds.append(pltpu.async_copy(u_hbm.at[idx_v.at[2 * b]],
                                            rows_v.at[b], gsems[b]))
            sds = []
            for b in range(KB):
                gds[b].wait()
                sds.append(pltpu.async_copy(rows_v.at[b],
                                            acc_sh.at[idx_v.at[2 * b + 1]],
                                            ssem, add=True))
            for d in sds:
                d.wait()
            return carry

        lax.fori_loop(0, nch // KB, body, 0)
        plsc.subcore_barrier()
        pltpu.sync_copy(
            acc_sh.at[pl.ds(s * RPT, RPT)], out_hbm.at[c, pl.ds(s * RPT, RPT)]
        )

    return prop


_prop128_edge = _make_prop(128, colsplit=False)
_prop128_col = _make_prop(128, colsplit=True)


def _dis_u0_body(d0_ref, d1_ref, x_ref, dis_ref, u0_ref):
    deg = d0_ref[...] + d1_ref[...] + 1.0
    dis = lax.rsqrt(deg)
    dis_ref[...] = dis
    u0_ref[...] = dis * x_ref[...]


def _dis_u0(d0, d1, x):
    grid = (N // BN,)
    return pl.pallas_call(
        _dis_u0_body,
        grid=grid,
        in_specs=[
            pl.BlockSpec((BN, 1), lambda i: (i, 0)),
            pl.BlockSpec((BN, 1), lambda i: (i, 0)),
            pl.BlockSpec((BN, NFEAT), lambda i: (i, 0)),
        ],
        out_specs=[
            pl.BlockSpec((BN, 1), lambda i: (i, 0)),
            pl.BlockSpec((BN, NFEAT), lambda i: (i, 0)),
        ],
        out_shape=[
            jax.ShapeDtypeStruct((N, 1), jnp.float32),
            jax.ShapeDtypeStruct((N, NFEAT), jnp.float32),
        ],
    )(d0, d1, x)


def _layer0_body(s0a_ref, s0b_ref, u0_ref, dis_ref, w_ref, sc_ref, bi_ref,
                 u1_ref):
    dis = dis_ref[...]
    z = dis * (s0a_ref[...] + s0b_ref[...] + u0_ref[...])
    y = jnp.dot(z, w_ref[...], preferred_element_type=jnp.float32)
    h = jnp.maximum(y * sc_ref[...] + bi_ref[...], 0.0)
    u1_ref[0] = dis * h


def _layer0(s0a, s0b, u0, dis, w0t, scale0, bias0):
    grid = (N // BN, 2)
    return pl.pallas_call(
        _layer0_body,
        grid=grid,
        in_specs=[
            pl.BlockSpec((BN, NFEAT), lambda i, j: (i, 0)),
            pl.BlockSpec((BN, NFEAT), lambda i, j: (i, 0)),
            pl.BlockSpec((BN, NFEAT), lambda i, j: (i, 0)),
            pl.BlockSpec((BN, 1), lambda i, j: (i, 0)),
            pl.BlockSpec((NFEAT, 128), lambda i, j: (0, j)),
            pl.BlockSpec((1, 128), lambda i, j: (0, j)),
            pl.BlockSpec((1, 128), lambda i, j: (0, j)),
        ],
        out_specs=pl.BlockSpec((1, BN, 128), lambda i, j: (j, i, 0)),
        out_shape=jax.ShapeDtypeStruct((2, N, 128), jnp.float32),
    )(s0a, s0b, u0, dis, w0t, scale0, bias0)


def _layer12_body(s1a_ref, s1b_ref, u1a_ref, u1b_ref, dis_ref, wa_ref, wb_ref,
                  sc_ref, bi_ref, w2_ref, z2_ref):
    dis = dis_ref[...]
    za = dis * (s1a_ref[...] + u1a_ref[...])
    zb = dis * (s1b_ref[...] + u1b_ref[...])
    y = jnp.dot(za, wa_ref[...], preferred_element_type=jnp.float32)
    y += jnp.dot(zb, wb_ref[...], preferred_element_type=jnp.float32)
    h2 = jnp.maximum(y * sc_ref[...] + bi_ref[...], 0.0)
    z2_ref[...] = dis * jnp.dot(h2, w2_ref[...],
                                preferred_element_type=jnp.float32)


def _layer12(s1a, s1b, u1a, u1b, dis, w1ta, w1tb, scale1, bias1, w2t):
    grid = (N // BN,)
    return pl.pallas_call(
        _layer12_body,
        grid=grid,
        in_specs=[
            pl.BlockSpec((BN, 128), lambda i: (i, 0)),
            pl.BlockSpec((BN, 128), lambda i: (i, 0)),
            pl.BlockSpec((BN, 128), lambda i: (i, 0)),
            pl.BlockSpec((BN, 128), lambda i: (i, 0)),
            pl.BlockSpec((BN, 1), lambda i: (i, 0)),
            pl.BlockSpec((128, NHID), lambda i: (0, 0)),
            pl.BlockSpec((128, NHID), lambda i: (0, 0)),
            pl.BlockSpec((1, NHID), lambda i: (0, 0)),
            pl.BlockSpec((1, NHID), lambda i: (0, 0)),
            pl.BlockSpec((NHID, 128), lambda i: (0, 0)),
        ],
        out_specs=pl.BlockSpec((BN, 128), lambda i: (i, 0)),
        out_shape=jax.ShapeDtypeStruct((N, 128), jnp.float32),
    )(s1a, s1b, u1a, u1b, dis, w1ta, w1tb, scale1, bias1, w2t)


def _final_body(s2a_ref, s2b_ref, z2_ref, dis_ref, b2_ref, out_ref):
    out_ref[...] = (
        dis_ref[...] * (s2a_ref[...] + s2b_ref[...] + z2_ref[...]) + b2_ref[...]
    )


def _final(s2a, s2b, z2, dis, b2):
    grid = (N // BN,)
    return pl.pallas_call(
        _final_body,
        grid=grid,
        in_specs=[
            pl.BlockSpec((BN, NCLASS), lambda i: (i, 0)),
            pl.BlockSpec((BN, NCLASS), lambda i: (i, 0)),
            pl.BlockSpec((BN, NCLASS), lambda i: (i, 0)),
            pl.BlockSpec((BN, 1), lambda i: (i, 0)),
            pl.BlockSpec((1, NCLASS), lambda i: (0, 0)),
        ],
        out_specs=pl.BlockSpec((BN, NCLASS), lambda i: (i, 0)),
        out_shape=jax.ShapeDtypeStruct((N, NCLASS), jnp.float32),
    )(s2a, s2b, z2, dis, b2)


def kernel(x, edge_idx, W0, b0, g0, be0, W1, b1, g1, be1, W2, b2):
    fill = EPAD - E
    rowp = jnp.concatenate([edge_idx[0], jnp.zeros((fill,), jnp.int32)])
    colp = jnp.concatenate([edge_idx[1], jnp.full((fill,), N, jnp.int32)])

    ones = jnp.ones((CH, DEGW), jnp.float32)
    zeros_deg = jnp.zeros((RPT, DEGW), jnp.float32)
    zeros128 = jnp.zeros((RPT, 128), jnp.float32)

    c0 = 1.0 / jnp.sqrt(jnp.float32(1.0 + 1e-5))
    scale0 = (g0 * c0)[None, :]
    bias0 = (b0 * g0 * c0 + be0)[None, :]
    scale1 = (g1 * c0)[None, :]
    bias1 = (b1 * g1 * c0 + be1)[None, :]
    w0t = W0.T
    w1t = W1.T
    w2t = jnp.pad(W2.T, ((0, 0), (0, 128 - NCLASS)))

    degp = _deg_sc(colp, ones, zeros_deg)
    d0 = degp[0, :N, :1]
    d1 = degp[1, :N, :1]
    dis, u0 = _dis_u0(d0, d1, x)

    s0 = _prop128_edge(u0, rowp, colp, zeros128)
    u1 = _layer0(s0[0, :N], s0[1, :N], u0, dis, w0t, scale0, bias0)

    s1 = _prop128_col(u1.reshape(2 * N, 128), rowp, colp, zeros128)
    z2 = _layer12(s1[0, :N], s1[1, :N], u1[0], u1[1], dis,
                  w1t[:128], w1t[128:], scale1, bias1, w2t)

    s2 = _prop128_edge(z2, rowp, colp, zeros128)
    return _final(s2[0, :N, :NCLASS], s2[1, :N, :NCLASS], z2[:, :NCLASS],
                  dis, b2[None, :])


# speedup vs baseline: 17.5083x; 2.3272x over previous
import functools

import jax
import jax.numpy as jnp
from jax import lax
from jax.experimental import pallas as pl
from jax.experimental.pallas import tpu as pltpu
from jax.experimental.pallas import tpu_sc as plsc

N = 10000
NFEAT = 128
NHID = 256
NCLASS = 64
E = 320000

NC, NS = 2, 16
NPAD = 10240
EPAD = 327680
CH = 128
RPT = NPAD // NS
DEGW = 16
BN = 2000


def _sc_mesh():
    return plsc.VectorSubcoreMesh(
        core_axis_name="c", subcore_axis_name="s", num_cores=NC, num_subcores=NS
    )


KB = 2
NCHUNK = EPAD // CH


@functools.partial(
    pl.kernel,
    out_type=jax.ShapeDtypeStruct((NC, NPAD, DEGW), jnp.float32),
    mesh=_sc_mesh(),
    scratch_types=[
        pltpu.VMEM((KB, CH), jnp.int32),
        pltpu.VMEM((CH, DEGW), jnp.float32),
        pltpu.VMEM_SHARED((NPAD, DEGW), jnp.float32),
        pltpu.SemaphoreType.DMA,
        pltpu.SemaphoreType.DMA,
    ],
)
def _deg_sc(colp_hbm, ones_hbm, zeros_hbm, out_hbm, idx_v, ones_v, acc_sh,
            isem, ssem):
    c = lax.axis_index("c")
    s = lax.axis_index("s")
    pltpu.sync_copy(zeros_hbm, acc_sh.at[pl.ds(s * RPT, RPT)])
    pltpu.sync_copy(ones_hbm, ones_v)
    plsc.subcore_barrier()
    nch = NCHUNK // (NC * NS)
    base = (c * NS + s) * nch

    def body(t, carry):
        ch0 = base + t * KB
        ids = [
            pltpu.async_copy(colp_hbm.at[pl.ds((ch0 + b) * CH, CH)],
                             idx_v.at[b], isem)
            for b in range(KB)
        ]
        for d in ids:
            d.wait()
        sds = [
            pltpu.async_copy(ones_v, acc_sh.at[idx_v.at[b]], ssem, add=True)
            for b in range(KB)
        ]
        for d in sds:
            d.wait()
        return carry

    lax.fori_loop(0, nch // KB, body, 0)
    plsc.subcore_barrier()
    pltpu.sync_copy(
        acc_sh.at[pl.ds(s * RPT, RPT)], out_hbm.at[c, pl.ds(s * RPT, RPT)]
    )


def _make_prop(width, colsplit):
    nch = NCHUNK // NS if colsplit else NCHUNK // (NC * NS)

    @functools.partial(
        pl.kernel,
        out_type=jax.ShapeDtypeStruct((NC, NPAD, width), jnp.float32),
        mesh=_sc_mesh(),
        scratch_types=[
            pltpu.VMEM((2 * KB, CH), jnp.int32),
            pltpu.VMEM((KB, CH, width), jnp.float32),
            pltpu.VMEM_SHARED((NPAD, width), jnp.float32),
            pltpu.SemaphoreType.DMA,
            pltpu.SemaphoreType.DMA,
            pltpu.SemaphoreType.DMA,
            pltpu.SemaphoreType.DMA,
        ],
    )
    def prop(u_hbm, rowp_hbm, colp_hbm, zeros_hbm, out_hbm,
             idx_v, rows_v, acc_sh, isem, gsem0, gsem1, ssem):
        gsems = [gsem0, gsem1]
        c = lax.axis_index("c")
        s = lax.axis_index("s")
        pltpu.sync_copy(zeros_hbm, acc_sh.at[pl.ds(s * RPT, RPT)])
        plsc.subcore_barrier()
        base = (s if colsplit else c * NS + s) * nch

        def body(t, carry):
            ch0 = base + t * KB
            ids = []
            for b in range(KB):
                off = (ch0 + b) * CH
                ids.append(pltpu.async_copy(rowp_hbm.at[pl.ds(off, CH)],
                                            idx_v.at[2 * b], isem))
                ids.append(pltpu.async_copy(colp_hbm.at[pl.ds(off, CH)],
                                            idx_v.at[2 * b + 1], isem))
            for d in ids:
                d.wait()
            gds = []
            for b in range(KB):
                if colsplit:
                    for i in range(CH // 16):
                        sl = pl.ds(i * 16, 16)
                        idx_v[2 * b, sl] = idx_v[2 * b, sl] + c * N
                gds.append(pltpu.async_copy(u_hbm.at[idx_v.at[2 * b]],
                                            rows_v.at[b], gsems[b]))
            sds = []
            for b in range(KB):
                gds[b].wait()
                sds.append(pltpu.async_copy(rows_v.at[b],
                                            acc_sh.at[idx_v.at[2 * b + 1]],
                                            ssem, add=True))
            for d in sds:
                d.wait()
            return carry

        lax.fori_loop(0, nch // KB, body, 0)
        plsc.subcore_barrier()
        pltpu.sync_copy(
            acc_sh.at[pl.ds(s * RPT, RPT)], out_hbm.at[c, pl.ds(s * RPT, RPT)]
        )

    return prop


_prop128_edge = _make_prop(128, colsplit=False)
_prop128_col = _make_prop(128, colsplit=True)


def _dis_u0_body(d0_ref, d1_ref, x_ref, dis_ref, u0_ref):
    deg = d0_ref[...] + d1_ref[...] + 1.0
    dis = lax.rsqrt(deg)
    dis_ref[...] = dis
    u0_ref[...] = dis * x_ref[...]


def _dis_u0(d0, d1, x):
    grid = (N // BN,)
    return pl.pallas_call(
        _dis_u0_body,
        grid=grid,
        in_specs=[
            pl.BlockSpec((BN, 1), lambda i: (i, 0)),
            pl.BlockSpec((BN, 1), lambda i: (i, 0)),
            pl.BlockSpec((BN, NFEAT), lambda i: (i, 0)),
        ],
        out_specs=[
            pl.BlockSpec((BN, 1), lambda i: (i, 0)),
            pl.BlockSpec((BN, NFEAT), lambda i: (i, 0)),
        ],
        out_shape=[
            jax.ShapeDtypeStruct((N, 1), jnp.float32),
            jax.ShapeDtypeStruct((N, NFEAT), jnp.float32),
        ],
    )(d0, d1, x)


def _layer0_body(s0a_ref, s0b_ref, u0_ref, dis_ref, w_ref, sc_ref, bi_ref,
                 u1_ref):
    dis = dis_ref[...]
    z = dis * (s0a_ref[...] + s0b_ref[...] + u0_ref[...])
    y = jnp.dot(z, w_ref[...], preferred_element_type=jnp.float32)
    h = jnp.maximum(y * sc_ref[...] + bi_ref[...], 0.0)
    u1_ref[0] = dis * h


def _layer0(s0a, s0b, u0, dis, w0t, scale0, bias0):
    grid = (N // BN, 2)
    return pl.pallas_call(
        _layer0_body,
        grid=grid,
        in_specs=[
            pl.BlockSpec((BN, NFEAT), lambda i, j: (i, 0)),
            pl.BlockSpec((BN, NFEAT), lambda i, j: (i, 0)),
            pl.BlockSpec((BN, NFEAT), lambda i, j: (i, 0)),
            pl.BlockSpec((BN, 1), lambda i, j: (i, 0)),
            pl.BlockSpec((NFEAT, 128), lambda i, j: (0, j)),
            pl.BlockSpec((1, 128), lambda i, j: (0, j)),
            pl.BlockSpec((1, 128), lambda i, j: (0, j)),
        ],
        out_specs=pl.BlockSpec((1, BN, 128), lambda i, j: (j, i, 0)),
        out_shape=jax.ShapeDtypeStruct((2, N, 128), jnp.float32),
    )(s0a, s0b, u0, dis, w0t, scale0, bias0)


def _layer12_body(s1a_ref, s1b_ref, u1a_ref, u1b_ref, dis_ref, wa_ref, wb_ref,
                  sc_ref, bi_ref, w2_ref, z2_ref):
    dis = dis_ref[...]
    za = dis * (s1a_ref[...] + u1a_ref[...])
    zb = dis * (s1b_ref[...] + u1b_ref[...])
    y = jnp.dot(za, wa_ref[...], preferred_element_type=jnp.float32)
    y += jnp.dot(zb, wb_ref[...], preferred_element_type=jnp.float32)
    h2 = jnp.maximum(y * sc_ref[...] + bi_ref[...], 0.0)
    z2_ref[...] = dis * jnp.dot(h2, w2_ref[...],
                                preferred_element_type=jnp.float32)


def _layer12(s1a, s1b, u1a, u1b, dis, w1ta, w1tb, scale1, bias1, w2t):
    grid = (N // BN,)
    return pl.pallas_call(
        _layer12_body,
        grid=grid,
        in_specs=[
            pl.BlockSpec((BN, 128), lambda i: (i, 0)),
            pl.BlockSpec((BN, 128), lambda i: (i, 0)),
            pl.BlockSpec((BN, 128), lambda i: (i, 0)),
            pl.BlockSpec((BN, 128), lambda i: (i, 0)),
            pl.BlockSpec((BN, 1), lambda i: (i, 0)),
            pl.BlockSpec((128, NHID), lambda i: (0, 0)),
            pl.BlockSpec((128, NHID), lambda i: (0, 0)),
            pl.BlockSpec((1, NHID), lambda i: (0, 0)),
            pl.BlockSpec((1, NHID), lambda i: (0, 0)),
            pl.BlockSpec((NHID, 128), lambda i: (0, 0)),
        ],
        out_specs=pl.BlockSpec((BN, 128), lambda i: (i, 0)),
        out_shape=jax.ShapeDtypeStruct((N, 128), jnp.float32),
    )(s1a, s1b, u1a, u1b, dis, w1ta, w1tb, scale1, bias1, w2t)


def _final_body(s2a_ref, s2b_ref, z2_ref, dis_ref, b2_ref, out_ref):
    out_ref[...] = (
        dis_ref[...] * (s2a_ref[...] + s2b_ref[...] + z2_ref[...]) + b2_ref[...]
    )


def _final(s2a, s2b, z2, dis, b2):
    grid = (N // BN,)
    return pl.pallas_call(
        _final_body,
        grid=grid,
        in_specs=[
            pl.BlockSpec((BN, NCLASS), lambda i: (i, 0)),
            pl.BlockSpec((BN, NCLASS), lambda i: (i, 0)),
            pl.BlockSpec((BN, NCLASS), lambda i: (i, 0)),
            pl.BlockSpec((BN, 1), lambda i: (i, 0)),
            pl.BlockSpec((1, NCLASS), lambda i: (0, 0)),
        ],
        out_specs=pl.BlockSpec((BN, NCLASS), lambda i: (i, 0)),
        out_shape=jax.ShapeDtypeStruct((N, NCLASS), jnp.float32),
    )(s2a, s2b, z2, dis, b2)


def kernel(x, edge_idx, W0, b0, g0, be0, W1, b1, g1, be1, W2, b2):
    fill = EPAD - E
    fidx = jnp.arange(fill, dtype=jnp.int32)
    rowp = jnp.concatenate([edge_idx[0], fidx % N])
    colp = jnp.concatenate([edge_idx[1], N + fidx % (NPAD - N)])

    ones = jnp.ones((CH, DEGW), jnp.float32)
    zeros_deg = jnp.zeros((RPT, DEGW), jnp.float32)
    zeros128 = jnp.zeros((RPT, 128), jnp.float32)

    c0 = 1.0 / jnp.sqrt(jnp.float32(1.0 + 1e-5))
    scale0 = (g0 * c0)[None, :]
    bias0 = (b0 * g0 * c0 + be0)[None, :]
    scale1 = (g1 * c0)[None, :]
    bias1 = (b1 * g1 * c0 + be1)[None, :]
    w0t = W0.T
    w1t = W1.T
    w2t = jnp.pad(W2.T, ((0, 0), (0, 128 - NCLASS)))

    degp = _deg_sc(colp, ones, zeros_deg)
    d0 = degp[0, :N, :1]
    d1 = degp[1, :N, :1]
    dis, u0 = _dis_u0(d0, d1, x)

    s0 = _prop128_edge(u0, rowp, colp, zeros128)
    u1 = _layer0(s0[0, :N], s0[1, :N], u0, dis, w0t, scale0, bias0)

    s1 = _prop128_col(u1.reshape(2 * N, 128), rowp, colp, zeros128)
    z2 = _layer12(s1[0, :N], s1[1, :N], u1[0], u1[1], dis,
                  w1t[:128], w1t[128:], scale1, bias1, w2t)

    s2 = _prop128_edge(z2, rowp, colp, zeros128)
    return _final(s2[0, :N, :NCLASS], s2[1, :N, :NCLASS], z2[:, :NCLASS],
                  dis, b2[None, :])


# speedup vs baseline: 18.0929x; 1.0334x over previous
import functools

import jax
import jax.numpy as jnp
from jax import lax
from jax.experimental import pallas as pl
from jax.experimental.pallas import tpu as pltpu
from jax.experimental.pallas import tpu_sc as plsc

N = 10000
NFEAT = 128
NHID = 256
NCLASS = 64
E = 320000

NC, NS = 2, 16
NPAD = 10240
EPAD = 327680
CH = 128
RPT = NPAD // NS
DEGW = 16
BN = 2000


def _sc_mesh():
    return plsc.VectorSubcoreMesh(
        core_axis_name="c", subcore_axis_name="s", num_cores=NC, num_subcores=NS
    )


KB = 2
NCHUNK = EPAD // CH


@functools.partial(
    pl.kernel,
    out_type=jax.ShapeDtypeStruct((NC, NPAD, DEGW), jnp.float32),
    mesh=_sc_mesh(),
    scratch_types=[
        pltpu.VMEM((KB, CH), jnp.int32),
        pltpu.VMEM((CH, DEGW), jnp.float32),
        pltpu.VMEM_SHARED((NPAD, DEGW), jnp.float32),
        pltpu.SemaphoreType.DMA,
        pltpu.SemaphoreType.DMA,
    ],
)
def _deg_sc(colp_hbm, ones_hbm, zeros_hbm, out_hbm, idx_v, ones_v, acc_sh,
            isem, ssem):
    c = lax.axis_index("c")
    s = lax.axis_index("s")
    pltpu.sync_copy(zeros_hbm, acc_sh.at[pl.ds(s * RPT, RPT)])
    pltpu.sync_copy(ones_hbm, ones_v)
    plsc.subcore_barrier()
    nch = NCHUNK // (NC * NS)
    base = (c * NS + s) * nch

    def body(t, carry):
        ch0 = base + t * KB
        ids = [
            pltpu.async_copy(colp_hbm.at[pl.ds((ch0 + b) * CH, CH)],
                             idx_v.at[b], isem)
            for b in range(KB)
        ]
        for d in ids:
            d.wait()
        sds = [
            pltpu.async_copy(ones_v, acc_sh.at[idx_v.at[b]], ssem, add=True)
            for b in range(KB)
        ]
        for d in sds:
            d.wait()
        return carry

    lax.fori_loop(0, nch // KB, body, 0)
    plsc.subcore_barrier()
    pltpu.sync_copy(
        acc_sh.at[pl.ds(s * RPT, RPT)], out_hbm.at[c, pl.ds(s * RPT, RPT)]
    )


def _make_prop(width, colsplit):
    nch = NCHUNK // NS if colsplit else NCHUNK // (NC * NS)

    @functools.partial(
        pl.kernel,
        out_type=jax.ShapeDtypeStruct((NC, NPAD, width), jnp.float32),
        mesh=_sc_mesh(),
        scratch_types=[
            pltpu.VMEM((8, CH), jnp.int32),
            pltpu.VMEM((2, CH, width), jnp.float32),
            pltpu.VMEM_SHARED((NPAD, width), jnp.float32),
        ] + [pltpu.SemaphoreType.DMA] * 5,
    )
    def prop(u_hbm, rowp_hbm, colp_hbm, zeros_hbm, out_hbm,
             idx_v, rows_v, acc_sh, isem, gsem0, gsem1, ssem0, ssem1):
        gsems = [gsem0, gsem1]
        ssems = [ssem0, ssem1]
        c = lax.axis_index("c")
        s = lax.axis_index("s")
        pltpu.sync_copy(zeros_hbm, acc_sh.at[pl.ds(s * RPT, RPT)])
        plsc.subcore_barrier()
        base = (s if colsplit else c * NS + s) * nch

        def body(t, carry):
            ch0 = base + 4 * t
            ids = []
            for j in range(4):
                off = (ch0 + j) * CH
                ids.append((
                    pltpu.async_copy(rowp_hbm.at[pl.ds(off, CH)],
                                     idx_v.at[2 * j], isem),
                    pltpu.async_copy(colp_hbm.at[pl.ds(off, CH)],
                                     idx_v.at[2 * j + 1], isem),
                ))

            def ready(j):
                ids[j][0].wait()
                ids[j][1].wait()
                if colsplit:
                    for i in range(CH // 16):
                        sl = pl.ds(i * 16, 16)
                        idx_v[2 * j, sl] = idx_v[2 * j, sl] + c * N

            def gather(j):
                return pltpu.async_copy(u_hbm.at[idx_v.at[2 * j]],
                                        rows_v.at[j % 2], gsems[j % 2])

            def scat(j):
                return pltpu.async_copy(rows_v.at[j % 2],
                                        acc_sh.at[idx_v.at[2 * j + 1]],
                                        ssems[j % 2], add=True)

            ready(0)
            g0 = gather(0)
            ready(1)
            g1 = gather(1)
            g0.wait()
            s0 = scat(0)
            g1.wait()
            s1 = scat(1)
            ready(2)
            s0.wait()
            g2 = gather(2)
            ready(3)
            g2.wait()
            s2 = scat(2)
            s1.wait()
            g3 = gather(3)
            g3.wait()
            s3 = scat(3)
            s2.wait()
            s3.wait()
            return carry

        lax.fori_loop(0, nch // 4, body, 0)
        plsc.subcore_barrier()
        pltpu.sync_copy(
            acc_sh.at[pl.ds(s * RPT, RPT)], out_hbm.at[c, pl.ds(s * RPT, RPT)]
        )

    return prop


_prop128_edge = _make_prop(128, colsplit=False)
_prop128_col = _make_prop(128, colsplit=True)


def _dis_u0_body(d0_ref, d1_ref, x_ref, dis_ref, u0_ref):
    deg = d0_ref[...] + d1_ref[...] + 1.0
    dis = lax.rsqrt(deg)
    dis_ref[...] = dis
    u0_ref[...] = dis * x_ref[...]


def _dis_u0(d0, d1, x):
    grid = (N // BN,)
    return pl.pallas_call(
        _dis_u0_body,
        grid=grid,
        in_specs=[
            pl.BlockSpec((BN, 1), lambda i: (i, 0)),
            pl.BlockSpec((BN, 1), lambda i: (i, 0)),
            pl.BlockSpec((BN, NFEAT), lambda i: (i, 0)),
        ],
        out_specs=[
            pl.BlockSpec((BN, 1), lambda i: (i, 0)),
            pl.BlockSpec((BN, NFEAT), lambda i: (i, 0)),
        ],
        out_shape=[
            jax.ShapeDtypeStruct((N, 1), jnp.float32),
            jax.ShapeDtypeStruct((N, NFEAT), jnp.float32),
        ],
    )(d0, d1, x)


def _layer0_body(s0a_ref, s0b_ref, u0_ref, dis_ref, w_ref, sc_ref, bi_ref,
                 u1_ref):
    dis = dis_ref[...]
    z = dis * (s0a_ref[...] + s0b_ref[...] + u0_ref[...])
    y = jnp.dot(z, w_ref[...], preferred_element_type=jnp.float32)
    h = jnp.maximum(y * sc_ref[...] + bi_ref[...], 0.0)
    u1_ref[0] = dis * h


def _layer0(s0a, s0b, u0, dis, w0t, scale0, bias0):
    grid = (N // BN, 2)
    return pl.pallas_call(
        _layer0_body,
        grid=grid,
        in_specs=[
            pl.BlockSpec((BN, NFEAT), lambda i, j: (i, 0)),
            pl.BlockSpec((BN, NFEAT), lambda i, j: (i, 0)),
            pl.BlockSpec((BN, NFEAT), lambda i, j: (i, 0)),
            pl.BlockSpec((BN, 1), lambda i, j: (i, 0)),
            pl.BlockSpec((NFEAT, 128), lambda i, j: (0, j)),
            pl.BlockSpec((1, 128), lambda i, j: (0, j)),
            pl.BlockSpec((1, 128), lambda i, j: (0, j)),
        ],
        out_specs=pl.BlockSpec((1, BN, 128), lambda i, j: (j, i, 0)),
        out_shape=jax.ShapeDtypeStruct((2, N, 128), jnp.float32),
    )(s0a, s0b, u0, dis, w0t, scale0, bias0)


def _layer12_body(s1a_ref, s1b_ref, u1a_ref, u1b_ref, dis_ref, wa_ref, wb_ref,
                  sc_ref, bi_ref, w2_ref, z2_ref):
    dis = dis_ref[...]
    za = dis * (s1a_ref[...] + u1a_ref[...])
    zb = dis * (s1b_ref[...] + u1b_ref[...])
    y = jnp.dot(za, wa_ref[...], preferred_element_type=jnp.float32)
    y += jnp.dot(zb, wb_ref[...], preferred_element_type=jnp.float32)
    h2 = jnp.maximum(y * sc_ref[...] + bi_ref[...], 0.0)
    z2_ref[...] = dis * jnp.dot(h2, w2_ref[...],
                                preferred_element_type=jnp.float32)


def _layer12(s1a, s1b, u1a, u1b, dis, w1ta, w1tb, scale1, bias1, w2t):
    grid = (N // BN,)
    return pl.pallas_call(
        _layer12_body,
        grid=grid,
        in_specs=[
            pl.BlockSpec((BN, 128), lambda i: (i, 0)),
            pl.BlockSpec((BN, 128), lambda i: (i, 0)),
            pl.BlockSpec((BN, 128), lambda i: (i, 0)),
            pl.BlockSpec((BN, 128), lambda i: (i, 0)),
            pl.BlockSpec((BN, 1), lambda i: (i, 0)),
            pl.BlockSpec((128, NHID), lambda i: (0, 0)),
            pl.BlockSpec((128, NHID), lambda i: (0, 0)),
            pl.BlockSpec((1, NHID), lambda i: (0, 0)),
            pl.BlockSpec((1, NHID), lambda i: (0, 0)),
            pl.BlockSpec((NHID, 128), lambda i: (0, 0)),
        ],
        out_specs=pl.BlockSpec((BN, 128), lambda i: (i, 0)),
        out_shape=jax.ShapeDtypeStruct((N, 128), jnp.float32),
    )(s1a, s1b, u1a, u1b, dis, w1ta, w1tb, scale1, bias1, w2t)


def _final_body(s2a_ref, s2b_ref, z2_ref, dis_ref, b2_ref, out_ref):
    out_ref[...] = (
        dis_ref[...] * (s2a_ref[...] + s2b_ref[...] + z2_ref[...]) + b2_ref[...]
    )


def _final(s2a, s2b, z2, dis, b2):
    grid = (N // BN,)
    return pl.pallas_call(
        _final_body,
        grid=grid,
        in_specs=[
            pl.BlockSpec((BN, NCLASS), lambda i: (i, 0)),
            pl.BlockSpec((BN, NCLASS), lambda i: (i, 0)),
            pl.BlockSpec((BN, NCLASS), lambda i: (i, 0)),
            pl.BlockSpec((BN, 1), lambda i: (i, 0)),
            pl.BlockSpec((1, NCLASS), lambda i: (0, 0)),
        ],
        out_specs=pl.BlockSpec((BN, NCLASS), lambda i: (i, 0)),
        out_shape=jax.ShapeDtypeStruct((N, NCLASS), jnp.float32),
    )(s2a, s2b, z2, dis, b2)


def kernel(x, edge_idx, W0, b0, g0, be0, W1, b1, g1, be1, W2, b2):
    fill = EPAD - E
    fidx = jnp.arange(fill + 2 * CH, dtype=jnp.int32)
    rowp = jnp.concatenate([edge_idx[0], fidx % N])
    colp = jnp.concatenate([edge_idx[1], N + fidx % (NPAD - N)])

    ones = jnp.ones((CH, DEGW), jnp.float32)
    zeros_deg = jnp.zeros((RPT, DEGW), jnp.float32)
    zeros128 = jnp.zeros((RPT, 128), jnp.float32)

    c0 = 1.0 / jnp.sqrt(jnp.float32(1.0 + 1e-5))
    scale0 = (g0 * c0)[None, :]
    bias0 = (b0 * g0 * c0 + be0)[None, :]
    scale1 = (g1 * c0)[None, :]
    bias1 = (b1 * g1 * c0 + be1)[None, :]
    w0t = W0.T
    w1t = W1.T
    w2t = jnp.pad(W2.T, ((0, 0), (0, 128 - NCLASS)))

    degp = _deg_sc(colp, ones, zeros_deg)
    d0 = degp[0, :N, :1]
    d1 = degp[1, :N, :1]
    dis, u0 = _dis_u0(d0, d1, x)

    s0 = _prop128_edge(u0, rowp, colp, zeros128)
    u1 = _layer0(s0[0, :N], s0[1, :N], u0, dis, w0t, scale0, bias0)

    s1 = _prop128_col(u1.reshape(2 * N, 128), rowp, colp, zeros128)
    z2 = _layer12(s1[0, :N], s1[1, :N], u1[0], u1[1], dis,
                  w1t[:128], w1t[128:], scale1, bias1, w2t)

    s2 = _prop128_edge(z2, rowp, colp, zeros128)
    return _final(s2[0, :N, :NCLASS], s2[1, :N, :NCLASS], z2[:, :NCLASS],
                  dis, b2[None, :])
